# baseline (speedup 1.0000x reference)
import functools
import jax
import jax.numpy as jnp
from jax import lax
from jax.experimental import pallas as pl
from jax.experimental.pallas import tpu as pltpu
from jax.experimental.pallas import tpu_sc as plsc

_N = 2048
_B = 8
_C = 16
_K = 10
_R = 512
_SWEEPS = 4
_JACOBI_ORDER = ((0, 2), (1, 2), (0, 1))


def _idx_body(xyz_ref, idx_ref):
    t = pl.program_id(1)
    P = xyz_ref[0]
    x, y, z = P[0:1, :], P[1:2, :], P[2:3, :]
    sq = x * x + y * y + z * z
    Prow = xyz_ref[0, :, pl.ds(t * _R, _R)]
    xr, yr, zr = Prow[0:1, :], Prow[1:2, :], Prow[2:3, :]
    sqr = xr * xr + yr * yr + zr * zr
    G = jax.lax.dot_general(P, Prow, (((0,), (0,)), ((), ())),
                            preferred_element_type=jnp.float32)
    d2 = (jnp.transpose(sq) + sqr) - 2.0 * G
    iota = lax.broadcasted_iota(jnp.int32, (_N, 1), 0).astype(jnp.float32)
    cur = d2
    rows = []
    for k in range(_K):
        thr = jnp.min(cur, axis=0, keepdims=True)
        hit = cur == thr
        ind = jnp.min(jnp.where(hit, iota, float(_N)), axis=0, keepdims=True)
        rows.append(ind)
        if k < _K - 1:
            cur = jnp.where(hit, jnp.inf, cur)
    rows += [rows[-1]] * 6
    idx_ref[0] = jnp.concatenate(rows, axis=0).astype(jnp.int32)


def _nn_indices(xyz):
    return pl.pallas_call(
        _idx_body,
        grid=(_C, _N // _R),
        in_specs=[pl.BlockSpec((1, 3, _N), lambda c, t: (c, 0, 0))],
        out_specs=pl.BlockSpec((1, 16, _R), lambda c, t: (c, 0, t)),
        out_shape=jax.ShapeDtypeStruct((_C, 16, _N), jnp.int32),
        compiler_params=pltpu.CompilerParams(
            dimension_semantics=("arbitrary", "arbitrary")),
    )(xyz)


_PPW = _C * _N // 32
_CHUNKS = _PPW // 16


def _cov_sc(xyz_flat, idx_flat):
    info = plsc.get_sparse_core_info()
    nc = info.num_cores

    @functools.partial(
        pl.kernel,
        mesh=plsc.VectorSubcoreMesh(core_axis_name="c", subcore_axis_name="s"),
        out_type=jax.ShapeDtypeStruct((_C * 8 * _N,), jnp.float32),
        compiler_params=pltpu.CompilerParams(needs_layout_passes=False),
        scratch_types=[
            pltpu.VMEM((_N,), jnp.float32),
            pltpu.VMEM((_N,), jnp.float32),
            pltpu.VMEM((_N,), jnp.float32),
            pltpu.VMEM((_K * _PPW,), jnp.int32),
            pltpu.VMEM((_K * 16,), jnp.float32),
            pltpu.VMEM((_K * 16,), jnp.float32),
            pltpu.VMEM((_K * 16,), jnp.float32),
            pltpu.VMEM((6 * _PPW,), jnp.float32),
        ],
    )
    def sc_cov(xyz_hbm, idx_hbm, out_hbm, x_v, y_v, z_v, idx_v,
               nb_x, nb_y, nb_z, out_v):
        wid = lax.axis_index("s") * nc + lax.axis_index("c")
        cloud = wid // 2
        half = wid % 2
        base = half * _PPW
        pltpu.sync_copy(xyz_hbm.at[pl.ds((cloud * 3 + 0) * _N, _N)], x_v)
        pltpu.sync_copy(xyz_hbm.at[pl.ds((cloud * 3 + 1) * _N, _N)], y_v)
        pltpu.sync_copy(xyz_hbm.at[pl.ds((cloud * 3 + 2) * _N, _N)], z_v)
        for jj in range(_K):
            pltpu.sync_copy(
                idx_hbm.at[pl.ds(cloud * 16 * _N + jj * _N + base, _PPW)],
                idx_v.at[pl.ds(jj * _PPW, _PPW)])

        def chunk(ch, carry):
            zero = jnp.zeros((16,), jnp.float32)
            sx = sy = sz = zero
            for jj in range(_K):
                iv = idx_v[pl.ds(jj * _PPW + ch * 16, 16)]
                nx = plsc.load_gather(x_v, [iv])
                ny = plsc.load_gather(y_v, [iv])
                nz = plsc.load_gather(z_v, [iv])
                nb_x[pl.ds(jj * 16, 16)] = nx
                nb_y[pl.ds(jj * 16, 16)] = ny
                nb_z[pl.ds(jj * 16, 16)] = nz
                sx = sx + nx
                sy = sy + ny
                sz = sz + nz
            mx = sx / float(_K)
            my = sy / float(_K)
            mz = sz / float(_K)
            s00 = s11 = s22 = s01 = s02 = s12 = zero
            for jj in range(_K):
                cx = nb_x[pl.ds(jj * 16, 16)] - mx
                cy = nb_y[pl.ds(jj * 16, 16)] - my
                cz = nb_z[pl.ds(jj * 16, 16)] - mz
                s00 = s00 + cx * cx
                s11 = s11 + cy * cy
                s22 = s22 + cz * cz
                s01 = s01 + cx * cy
                s02 = s02 + cx * cz
                s12 = s12 + cy * cz
            out_v[pl.ds(0 * _PPW + ch * 16, 16)] = s00 / float(_K)
            out_v[pl.ds(1 * _PPW + ch * 16, 16)] = s11 / float(_K)
            out_v[pl.ds(2 * _PPW + ch * 16, 16)] = s22 / float(_K)
            out_v[pl.ds(3 * _PPW + ch * 16, 16)] = s01 / float(_K)
            out_v[pl.ds(4 * _PPW + ch * 16, 16)] = s02 / float(_K)
            out_v[pl.ds(5 * _PPW + ch * 16, 16)] = s12 / float(_K)
            return carry

        lax.fori_loop(0, _CHUNKS, chunk, 0)
        for comp in range(6):
            pltpu.sync_copy(
                out_v.at[pl.ds(comp * _PPW, _PPW)],
                out_hbm.at[pl.ds(cloud * 8 * _N + comp * _N + base, _PPW)])

    return sc_cov(xyz_flat, idx_flat)


def _rotate(A, V, p, q):
    app, aqq, apq = A[p][p], A[q][q], A[p][q]
    tau = (aqq - app) / (2.0 * apq)
    t = jnp.sign(tau) / (jnp.abs(tau) + jnp.sqrt(1.0 + tau * tau))
    t = jnp.where(tau == 0.0, 1.0, t)
    c = 1.0 / jnp.sqrt(1.0 + t * t)
    s = t * c
    zero = apq == 0.0
    c = jnp.where(zero, 1.0, c)
    s = jnp.where(zero, 0.0, s)
    for j in range(3):
        ap, aq = A[p][j], A[q][j]
        A[p][j] = c * ap - s * aq
        A[q][j] = s * ap + c * aq
    for i in range(3):
        ap, aq = A[i][p], A[i][q]
        A[i][p] = c * ap - s * aq
        A[i][q] = s * ap + c * aq
    for i in range(3):
        vp, vq = V[i][p], V[i][q]
        V[i][p] = c * vp - s * vq
        V[i][q] = s * vp + c * vq


def _normal_from_cov(cov):
    a00, a11, a22 = cov[0:1, :], cov[1:2, :], cov[2:3, :]
    a01, a02, a12 = cov[3:4, :], cov[4:5, :], cov[5:6, :]
    A = [[a00, a01, a02], [a01, a11, a12], [a02, a12, a22]]
    one = jnp.ones_like(a00)
    nil = jnp.zeros_like(a00)
    V = [[one, nil, nil], [nil, one, nil], [nil, nil, one]]
    for _ in range(_SWEEPS):
        for (p, q) in _JACOBI_ORDER:
            _rotate(A, V, p, q)
    d0, d1, d2 = A[0][0], A[1][1], A[2][2]
    b1 = d1 < d0
    best = jnp.where(b1, d1, d0)
    n = [jnp.where(b1, V[i][1], V[i][0]) for i in range(3)]
    b2 = d2 < best
    n = [jnp.where(b2, V[i][2], n[i]) for i in range(3)]
    norm = jnp.sqrt(n[0] * n[0] + n[1] * n[1] + n[2] * n[2]) + 1e-12
    return n[0] / norm, n[1] / norm, n[2] / norm


def _loss_body(covp_ref, covg_ref, out_ref):
    b = pl.program_id(0)
    p0, p1, p2 = _normal_from_cov(covp_ref[0])
    g0, g1, g2 = _normal_from_cov(covg_ref[0])
    e0, e1, e2 = p0 - g0, p1 - g1, p2 - g2
    ssq = jnp.sum(e0 * e0 + e1 * e1 + e2 * e2, keepdims=True)

    @pl.when(b == 0)
    def _():
        out_ref[:, :] = jnp.zeros_like(ssq)

    out_ref[:, :] += ssq / float(_B * 3 * _N)


def _loss(cov):
    return pl.pallas_call(
        _loss_body,
        grid=(_B,),
        in_specs=[
            pl.BlockSpec((1, 8, _N), lambda b: (b, 0, 0)),
            pl.BlockSpec((1, 8, _N), lambda b: (b + _B, 0, 0)),
        ],
        out_specs=pl.BlockSpec((1, 1), lambda b: (0, 0)),
        out_shape=jax.ShapeDtypeStruct((1, 1), jnp.float32),
        compiler_params=pltpu.CompilerParams(
            dimension_semantics=("arbitrary",)),
    )(cov, cov)


def kernel(pred, gt):
    xyz = jnp.concatenate([pred, gt], axis=0)
    idx = _nn_indices(xyz)
    cov_flat = _cov_sc(xyz.reshape(-1), idx.reshape(-1))
    cov = cov_flat.reshape(_C, 8, _N)
    return _loss(cov)[0, 0]


# speedup vs baseline: 1.0551x; 1.0551x over previous
import functools
import jax
import jax.numpy as jnp
from jax import lax
from jax.experimental import pallas as pl
from jax.experimental.pallas import tpu as pltpu
from jax.experimental.pallas import tpu_sc as plsc

_N = 2048
_B = 8
_C = 16
_K = 10
_R = 1024
_SWEEPS = 4
_JACOBI_ORDER = ((0, 2), (1, 2), (0, 1))


def _idx_body(xyz_ref, idx_ref):
    t = pl.program_id(1)
    P = xyz_ref[0]
    x, y, z = P[0:1, :], P[1:2, :], P[2:3, :]
    sq = x * x + y * y + z * z
    Prow = xyz_ref[0, :, pl.ds(t * _R, _R)]
    xr, yr, zr = Prow[0:1, :], Prow[1:2, :], Prow[2:3, :]
    sqr = xr * xr + yr * yr + zr * zr
    G = jax.lax.dot_general(P, Prow, (((0,), (0,)), ((), ())),
                            preferred_element_type=jnp.float32)
    d2 = (jnp.transpose(sq) + sqr) - 2.0 * G
    iota = lax.broadcasted_iota(jnp.int32, (_N, 1), 0).astype(jnp.float32)
    rows = []
    thr = None
    for k in range(_K):
        v = d2 if k == 0 else jnp.where(d2 > thr, d2, jnp.inf)
        thr = jnp.min(v, axis=0, keepdims=True)
        hit = v == thr
        ind = jnp.min(jnp.where(hit, iota, float(_N)), axis=0, keepdims=True)
        rows.append(ind)
    rows += [rows[-1]] * 6
    idx_ref[0] = jnp.concatenate(rows, axis=0).astype(jnp.int32)


def _nn_indices(xyz):
    return pl.pallas_call(
        _idx_body,
        grid=(_C, _N // _R),
        in_specs=[pl.BlockSpec((1, 3, _N), lambda c, t: (c, 0, 0))],
        out_specs=pl.BlockSpec((1, 16, _R), lambda c, t: (c, 0, t)),
        out_shape=jax.ShapeDtypeStruct((_C, 16, _N), jnp.int32),
        compiler_params=pltpu.CompilerParams(
            dimension_semantics=("arbitrary", "arbitrary")),
    )(xyz)


_PPW = _C * _N // 32
_CHUNKS = _PPW // 16


def _cov_sc(xyz_flat, idx_flat):
    info = plsc.get_sparse_core_info()
    nc = info.num_cores

    @functools.partial(
        pl.kernel,
        mesh=plsc.VectorSubcoreMesh(core_axis_name="c", subcore_axis_name="s"),
        out_type=jax.ShapeDtypeStruct((_C * 8 * _N,), jnp.float32),
        compiler_params=pltpu.CompilerParams(needs_layout_passes=False),
        scratch_types=[
            pltpu.VMEM((_N,), jnp.float32),
            pltpu.VMEM((_N,), jnp.float32),
            pltpu.VMEM((_N,), jnp.float32),
            pltpu.VMEM((_K * _PPW,), jnp.int32),
            pltpu.VMEM((_K * 16,), jnp.float32),
            pltpu.VMEM((_K * 16,), jnp.float32),
            pltpu.VMEM((_K * 16,), jnp.float32),
            pltpu.VMEM((6 * _PPW,), jnp.float32),
        ],
    )
    def sc_cov(xyz_hbm, idx_hbm, out_hbm, x_v, y_v, z_v, idx_v,
               nb_x, nb_y, nb_z, out_v):
        wid = lax.axis_index("s") * nc + lax.axis_index("c")
        cloud = wid // 2
        half = wid % 2
        base = half * _PPW
        pltpu.sync_copy(xyz_hbm.at[pl.ds((cloud * 3 + 0) * _N, _N)], x_v)
        pltpu.sync_copy(xyz_hbm.at[pl.ds((cloud * 3 + 1) * _N, _N)], y_v)
        pltpu.sync_copy(xyz_hbm.at[pl.ds((cloud * 3 + 2) * _N, _N)], z_v)
        for jj in range(_K):
            pltpu.sync_copy(
                idx_hbm.at[pl.ds(cloud * 16 * _N + jj * _N + base, _PPW)],
                idx_v.at[pl.ds(jj * _PPW, _PPW)])

        def chunk(ch, carry):
            zero = jnp.zeros((16,), jnp.float32)
            sx = sy = sz = zero
            for jj in range(_K):
                iv = idx_v[pl.ds(jj * _PPW + ch * 16, 16)]
                nx = plsc.load_gather(x_v, [iv])
                ny = plsc.load_gather(y_v, [iv])
                nz = plsc.load_gather(z_v, [iv])
                nb_x[pl.ds(jj * 16, 16)] = nx
                nb_y[pl.ds(jj * 16, 16)] = ny
                nb_z[pl.ds(jj * 16, 16)] = nz
                sx = sx + nx
                sy = sy + ny
                sz = sz + nz
            mx = sx / float(_K)
            my = sy / float(_K)
            mz = sz / float(_K)
            s00 = s11 = s22 = s01 = s02 = s12 = zero
            for jj in range(_K):
                cx = nb_x[pl.ds(jj * 16, 16)] - mx
                cy = nb_y[pl.ds(jj * 16, 16)] - my
                cz = nb_z[pl.ds(jj * 16, 16)] - mz
                s00 = s00 + cx * cx
                s11 = s11 + cy * cy
                s22 = s22 + cz * cz
                s01 = s01 + cx * cy
                s02 = s02 + cx * cz
                s12 = s12 + cy * cz
            out_v[pl.ds(0 * _PPW + ch * 16, 16)] = s00 / float(_K)
            out_v[pl.ds(1 * _PPW + ch * 16, 16)] = s11 / float(_K)
            out_v[pl.ds(2 * _PPW + ch * 16, 16)] = s22 / float(_K)
            out_v[pl.ds(3 * _PPW + ch * 16, 16)] = s01 / float(_K)
            out_v[pl.ds(4 * _PPW + ch * 16, 16)] = s02 / float(_K)
            out_v[pl.ds(5 * _PPW + ch * 16, 16)] = s12 / float(_K)
            return carry

        lax.fori_loop(0, _CHUNKS, chunk, 0)
        for comp in range(6):
            pltpu.sync_copy(
                out_v.at[pl.ds(comp * _PPW, _PPW)],
                out_hbm.at[pl.ds(cloud * 8 * _N + comp * _N + base, _PPW)])

    return sc_cov(xyz_flat, idx_flat)


def _rotate(A, V, p, q):
    app, aqq, apq = A[p][p], A[q][q], A[p][q]
    tau = (aqq - app) / (2.0 * apq)
    t = jnp.sign(tau) / (jnp.abs(tau) + jnp.sqrt(1.0 + tau * tau))
    t = jnp.where(tau == 0.0, 1.0, t)
    c = 1.0 / jnp.sqrt(1.0 + t * t)
    s = t * c
    zero = apq == 0.0
    c = jnp.where(zero, 1.0, c)
    s = jnp.where(zero, 0.0, s)
    for j in range(3):
        ap, aq = A[p][j], A[q][j]
        A[p][j] = c * ap - s * aq
        A[q][j] = s * ap + c * aq
    for i in range(3):
        ap, aq = A[i][p], A[i][q]
        A[i][p] = c * ap - s * aq
        A[i][q] = s * ap + c * aq
    for i in range(3):
        vp, vq = V[i][p], V[i][q]
        V[i][p] = c * vp - s * vq
        V[i][q] = s * vp + c * vq


def _normal_from_cov(cov):
    a00, a11, a22 = cov[0:1, :], cov[1:2, :], cov[2:3, :]
    a01, a02, a12 = cov[3:4, :], cov[4:5, :], cov[5:6, :]
    A = [[a00, a01, a02], [a01, a11, a12], [a02, a12, a22]]
    one = jnp.ones_like(a00)
    nil = jnp.zeros_like(a00)
    V = [[one, nil, nil], [nil, one, nil], [nil, nil, one]]
    for _ in range(_SWEEPS):
        for (p, q) in _JACOBI_ORDER:
            _rotate(A, V, p, q)
    d0, d1, d2 = A[0][0], A[1][1], A[2][2]
    b1 = d1 < d0
    best = jnp.where(b1, d1, d0)
    n = [jnp.where(b1, V[i][1], V[i][0]) for i in range(3)]
    b2 = d2 < best
    n = [jnp.where(b2, V[i][2], n[i]) for i in range(3)]
    norm = jnp.sqrt(n[0] * n[0] + n[1] * n[1] + n[2] * n[2]) + 1e-12
    return n[0] / norm, n[1] / norm, n[2] / norm


def _loss_body(covp_ref, covg_ref, out_ref):
    b = pl.program_id(0)
    p0, p1, p2 = _normal_from_cov(covp_ref[0])
    g0, g1, g2 = _normal_from_cov(covg_ref[0])
    e0, e1, e2 = p0 - g0, p1 - g1, p2 - g2
    ssq = jnp.sum(e0 * e0 + e1 * e1 + e2 * e2, keepdims=True)

    @pl.when(b == 0)
    def _():
        out_ref[:, :] = jnp.zeros_like(ssq)

    out_ref[:, :] += ssq / float(_B * 3 * _N)


def _loss(cov):
    return pl.pallas_call(
        _loss_body,
        grid=(_B,),
        in_specs=[
            pl.BlockSpec((1, 8, _N), lambda b: (b, 0, 0)),
            pl.BlockSpec((1, 8, _N), lambda b: (b + _B, 0, 0)),
        ],
        out_specs=pl.BlockSpec((1, 1), lambda b: (0, 0)),
        out_shape=jax.ShapeDtypeStruct((1, 1), jnp.float32),
        compiler_params=pltpu.CompilerParams(
            dimension_semantics=("arbitrary",)),
    )(cov, cov)


def kernel(pred, gt):
    xyz = jnp.concatenate([pred, gt], axis=0)
    idx = _nn_indices(xyz)
    cov_flat = _cov_sc(xyz.reshape(-1), idx.reshape(-1))
    cov = cov_flat.reshape(_C, 8, _N)
    return _loss(cov)[0, 0]
